# Initial kernel scaffold; baseline (speedup 1.0000x reference)
import jax
import jax.numpy as jnp
from jax.experimental import pallas as pl


def kernel(coords, features, neighbor_indices, W, b, gamma, beta, training):
    raise NotImplementedError("write your pallas kernel here")


# speedup vs baseline: 13.8293x; 13.8293x over previous
import functools

import jax
import jax.numpy as jnp
from jax import lax
from jax.experimental import pallas as pl
from jax.experimental.pallas import tpu as pltpu
from jax.experimental.pallas import tpu_sc as plsc

B, N, K, D, DOUT = 4, 8192, 16, 64, 64
E = B * N * K
BN_EPS = 1e-6
LEAKY_SLOPE = 0.2
CPAD = 8
CH = 128
EBLK = 2048


def _sc_gather(feats_flat, coords_pad, gidx):
    info = plsc.get_sparse_core_info()
    nwork = info.num_cores * info.num_subcores
    epw = E // nwork
    nch = epw // CH
    idx3 = gidx.reshape(nwork, nch, CH)
    mesh = plsc.VectorSubcoreMesh(core_axis_name="c", subcore_axis_name="s")

    @functools.partial(
        pl.kernel,
        mesh=mesh,
        compiler_params=pltpu.CompilerParams(use_tc_tiling_on_sc=False),
        out_type=[
            jax.ShapeDtypeStruct((E, D), jnp.float32),
            jax.ShapeDtypeStruct((E, CPAD), jnp.float32),
        ],
        scratch_types=[
            pltpu.VMEM((nch, CH), jnp.int32),
            pltpu.VMEM((CH, D), jnp.float32),
            pltpu.VMEM((CH, CPAD), jnp.float32),
            pltpu.SemaphoreType.DMA,
            pltpu.SemaphoreType.DMA,
        ],
    )
    def k(feats_hbm, coords_hbm, idx_hbm, nf_hbm, ncrd_hbm,
          idx_v, rows_v, crd_v, sem_f, sem_c):
        wid = lax.axis_index("s") * info.num_cores + lax.axis_index("c")
        wbase = wid * epw
        pltpu.sync_copy(idx_hbm.at[wid], idx_v)

        def body(c, carry):
            cp_f = pltpu.async_copy(feats_hbm.at[idx_v.at[c]], rows_v, sem_f)
            cp_c = pltpu.async_copy(coords_hbm.at[idx_v.at[c]], crd_v, sem_c)
            cp_f.wait()
            cp_c.wait()
            off = wbase + c * CH
            pltpu.sync_copy(rows_v, nf_hbm.at[pl.ds(off, CH)])
            pltpu.sync_copy(crd_v, ncrd_hbm.at[pl.ds(off, CH)])
            return carry

        lax.fori_loop(0, nch, body, 0)

    return k(feats_flat, coords_pad, idx3)


def _tc_body(nf_ref, nc_ref, ec_ref, wa_ref, wc_ref, b_ref,
             out_ref, rel_ref):
    nf = nf_ref[...]
    ncr = nc_ref[...]
    ec = ec_ref[...]
    rel = ec - ncr
    d2 = jnp.sum(rel * rel, axis=1, keepdims=True) + 1e-12
    dist = jnp.sqrt(d2)
    lane = lax.broadcasted_iota(jnp.int32, (1, CPAD), 1)
    onehot3 = (lane == 3).astype(jnp.float32)
    ecd = ec + (jnp.broadcast_to(dist, (EBLK, CPAD))
                * jnp.broadcast_to(onehot3, (EBLK, CPAD)))
    x = (jnp.dot(ecd, wa_ref[...], preferred_element_type=jnp.float32)
         + jnp.dot(ncr, wc_ref[...], preferred_element_type=jnp.float32)
         + jnp.broadcast_to(b_ref[...], (EBLK, DOUT)))
    y = jnp.maximum(x, LEAKY_SLOPE * x)
    out_ref[...] = jnp.concatenate([nf, y], axis=1)
    rel_ref[...] = y


def _tc_dense(nbr_feats, nbr_coords, ext_coords, wa, wc, beff):
    grid = (E // EBLK,)
    return pl.pallas_call(
        _tc_body,
        grid=grid,
        in_specs=[
            pl.BlockSpec((EBLK, D), lambda i: (i, 0)),
            pl.BlockSpec((EBLK, CPAD), lambda i: (i, 0)),
            pl.BlockSpec((EBLK, CPAD), lambda i: (i, 0)),
            pl.BlockSpec((CPAD, DOUT), lambda i: (0, 0)),
            pl.BlockSpec((CPAD, DOUT), lambda i: (0, 0)),
            pl.BlockSpec((1, DOUT), lambda i: (0, 0)),
        ],
        out_specs=[
            pl.BlockSpec((EBLK, D + DOUT), lambda i: (i, 0)),
            pl.BlockSpec((EBLK, DOUT), lambda i: (i, 0)),
        ],
        out_shape=[
            jax.ShapeDtypeStruct((E, D + DOUT), jnp.float32),
            jax.ShapeDtypeStruct((E, DOUT), jnp.float32),
        ],
    )(nbr_feats, nbr_coords, ext_coords, wa, wc, beff)


def kernel(coords, features, neighbor_indices, W, b, gamma, beta, training):
    del training
    feats_flat = features.reshape(B * N, D)
    cpad = jnp.pad(coords, ((0, 0), (0, 0), (0, CPAD - 3)))
    coords_flat = cpad.reshape(B * N, CPAD)
    ext_flat = jnp.broadcast_to(cpad[:, :, None, :], (B, N, K, CPAD)).reshape(E, CPAD)
    gidx = (neighbor_indices
            + (jnp.arange(B, dtype=jnp.int32) * N)[:, None, None]).reshape(E)

    scale = gamma / jnp.sqrt(1.0 + BN_EPS)
    w_eff = W * scale[None, :]
    beff = (b * scale + beta).reshape(1, DOUT)
    wa = jnp.concatenate([w_eff[4:7] + w_eff[1:4], w_eff[0:1],
                          jnp.zeros((CPAD - 4, DOUT), jnp.float32)], axis=0)
    wc = jnp.concatenate([w_eff[7:10] - w_eff[1:4],
                          jnp.zeros((CPAD - 3, DOUT), jnp.float32)], axis=0)

    nbr_feats, nbr_coords = _sc_gather(feats_flat, coords_flat, gidx)

    out_flat, rel_flat = _tc_dense(nbr_feats, nbr_coords, ext_flat,
                                   wa, wc, beff)
    return (out_flat.reshape(B, N, K, D + DOUT),
            rel_flat.reshape(B, N, K, DOUT))
